# baseline (speedup 1.0000x reference)
import jax
import jax.numpy as jnp
from jax import lax
from jax.experimental import pallas as pl
from jax.experimental.pallas import tpu as pltpu
from jax.experimental.pallas import tpu_sc as plsc

N_NODES = 10000
N_EDGES = 320000
D = 128

NC = 2
NS = 16
NW = NC * NS

E_PER_W = N_EDGES // NW
CHUNK = 80
STEPS = E_PER_W // CHUNK
NBUF = 3
GROUPS = (STEPS - 2) // NBUF

STG = 16
ROWS_T = 624
ROWS_LAST = N_NODES - 15 * ROWS_T


def _sc_body(x_hbm, eidx_hbm, out_hbm,
             idx_src, idx_dst, bufa, xneg_spm, sem_a, sem_w):
    cid = lax.axis_index("c")
    sid = lax.axis_index("s")
    wid = sid * NC + cid
    base = wid * E_PER_W

    rowbase = sid * ROWS_T
    nchunks = jnp.where(sid == NS - 1, ROWS_LAST // STG, ROWS_T // STG)
    tmp = bufa.at[pl.ds(0, STG)]

    def stage(c, carry):
        r = rowbase + c * STG
        pltpu.sync_copy(x_hbm.at[pl.ds(r, STG)], tmp)
        for row in range(STG):
            for jj in range(D // 16):
                s = pl.ds(jj * 16, 16)
                bufa[row, s] = -bufa[row, s]
        pltpu.sync_copy(tmp, xneg_spm.at[pl.ds(r, STG)])
        return carry

    lax.fori_loop(0, nchunks, stage, 0)
    plsc.subcore_barrier()

    pltpu.sync_copy(eidx_hbm.at[pl.ds(base, E_PER_W)], idx_src)
    pltpu.sync_copy(eidx_hbm.at[pl.ds(N_EDGES + base, E_PER_W)], idx_dst)

    def do_group(g, first):
        ga = []
        for b in range(NBUF):
            off = (g * NBUF + b) * CHUNK
            a = bufa.at[pl.ds(b * CHUNK, CHUNK)]
            if not first:
                pltpu.make_async_copy(
                    a, out_hbm.at[pl.ds(base + off, CHUNK)], sem_w[b]).wait()
            ga.append(pltpu.async_copy(
                x_hbm.at[idx_src.at[pl.ds(off, CHUNK)]], a, sem_a[b]))
        gd = []
        for b in range(NBUF):
            off = (g * NBUF + b) * CHUNK
            a = bufa.at[pl.ds(b * CHUNK, CHUNK)]
            ga[b].wait()
            gd.append(pltpu.async_copy(
                xneg_spm.at[idx_dst.at[pl.ds(off, CHUNK)]], a, sem_a[b],
                add=True))
        for b in range(NBUF):
            off = (g * NBUF + b) * CHUNK
            a = bufa.at[pl.ds(b * CHUNK, CHUNK)]
            gd[b].wait()
            pltpu.async_copy(a, out_hbm.at[pl.ds(base + off, CHUNK)],
                             sem_w[b])

    do_group(0, True)
    lax.fori_loop(1, GROUPS, lambda g, cr: (do_group(g, False), cr)[1], 0)

    for b in range(STEPS - NBUF * GROUPS):
        c = NBUF * GROUPS + b
        off = c * CHUNK
        a = bufa.at[pl.ds(b * CHUNK, CHUNK)]
        pltpu.make_async_copy(
            a, out_hbm.at[pl.ds(base + off, CHUNK)], sem_w[b]).wait()
        pltpu.async_copy(
            x_hbm.at[idx_src.at[pl.ds(off, CHUNK)]], a, sem_a[b]).wait()
        pltpu.async_copy(
            xneg_spm.at[idx_dst.at[pl.ds(off, CHUNK)]], a, sem_a[b],
            add=True).wait()
        pltpu.async_copy(a, out_hbm.at[pl.ds(base + off, CHUNK)], sem_w[b])

    for b in range(NBUF):
        a = bufa.at[pl.ds(b * CHUNK, CHUNK)]
        pltpu.make_async_copy(
            a, out_hbm.at[pl.ds(base + b * CHUNK, CHUNK)], sem_w[b]).wait()


@jax.jit
def kernel(x, edge_index):
    eidx = edge_index.reshape(-1)

    mesh = plsc.VectorSubcoreMesh(core_axis_name="c", subcore_axis_name="s")
    out = pl.kernel(
        _sc_body,
        out_type=jax.ShapeDtypeStruct((N_EDGES, D), jnp.float32),
        mesh=mesh,
        scratch_types=[
            pltpu.VMEM((E_PER_W,), jnp.int32),
            pltpu.VMEM((E_PER_W,), jnp.int32),
            pltpu.VMEM((NBUF * CHUNK, D), jnp.float32),
            pltpu.VMEM_SHARED((N_NODES, D), jnp.float32),
            [pltpu.SemaphoreType.DMA] * NBUF,
            [pltpu.SemaphoreType.DMA] * NBUF,
        ],
    )(x, eidx)
    return out


# speedup vs baseline: 1.0437x; 1.0437x over previous
import jax
import jax.numpy as jnp
from jax import lax
from jax.experimental import pallas as pl
from jax.experimental.pallas import tpu as pltpu
from jax.experimental.pallas import tpu_sc as plsc

N_NODES = 10000
N_EDGES = 320000
D = 128

NC = 2
NS = 16
NW = NC * NS

E_PER_W = N_EDGES // NW
CHUNK = 80
STEPS = E_PER_W // CHUNK
NBUF = 3
GROUPS = (STEPS - 2) // NBUF

STG = 16
ROWS_T = 624
ROWS_LAST = N_NODES - 15 * ROWS_T


def _sc_body(x_hbm, eidx_hbm, out_hbm,
             idx_src, idx_dst, bufa, xneg_spm, sem_a, sem_w):
    cid = lax.axis_index("c")
    sid = lax.axis_index("s")
    wid = sid * NC + cid
    base = wid * E_PER_W

    ic1 = pltpu.async_copy(eidx_hbm.at[pl.ds(base, E_PER_W)], idx_src,
                           sem_w[0])
    ic2 = pltpu.async_copy(eidx_hbm.at[pl.ds(N_EDGES + base, E_PER_W)],
                           idx_dst, sem_w[1])

    rowbase = sid * ROWS_T
    nchunks = jnp.where(sid == NS - 1, ROWS_LAST // STG, ROWS_T // STG)
    tmp = bufa.at[pl.ds(0, STG)]

    def stage(c, carry):
        r = rowbase + c * STG
        pltpu.sync_copy(x_hbm.at[pl.ds(r, STG)], tmp)
        for row in range(STG):
            for jj in range(D // 16):
                s = pl.ds(jj * 16, 16)
                bufa[row, s] = -bufa[row, s]
        pltpu.sync_copy(tmp, xneg_spm.at[pl.ds(r, STG)])
        return carry

    lax.fori_loop(0, nchunks, stage, 0)
    plsc.subcore_barrier()
    ic1.wait()
    ic2.wait()

    def do_group(g, first):
        ga = []
        for b in range(NBUF):
            off = (g * NBUF + b) * CHUNK
            a = bufa.at[pl.ds(b * CHUNK, CHUNK)]
            if not first:
                pltpu.make_async_copy(
                    a, out_hbm.at[pl.ds(base + off, CHUNK)], sem_w[b]).wait()
            ga.append(pltpu.async_copy(
                x_hbm.at[idx_src.at[pl.ds(off, CHUNK)]], a, sem_a[b]))
        for b in range(NBUF):
            off = (g * NBUF + b) * CHUNK
            a = bufa.at[pl.ds(b * CHUNK, CHUNK)]
            ga[b].wait()
            pltpu.async_copy(
                xneg_spm.at[idx_dst.at[pl.ds(off, CHUNK)]], a, sem_a[b],
                add=True).wait()
            pltpu.async_copy(a, out_hbm.at[pl.ds(base + off, CHUNK)],
                             sem_w[b])

    do_group(0, True)
    lax.fori_loop(1, GROUPS, lambda g, cr: (do_group(g, False), cr)[1], 0)

    for b in range(STEPS - NBUF * GROUPS):
        c = NBUF * GROUPS + b
        off = c * CHUNK
        a = bufa.at[pl.ds(b * CHUNK, CHUNK)]
        pltpu.make_async_copy(
            a, out_hbm.at[pl.ds(base + off, CHUNK)], sem_w[b]).wait()
        pltpu.async_copy(
            x_hbm.at[idx_src.at[pl.ds(off, CHUNK)]], a, sem_a[b]).wait()
        pltpu.async_copy(
            xneg_spm.at[idx_dst.at[pl.ds(off, CHUNK)]], a, sem_a[b],
            add=True).wait()
        pltpu.async_copy(a, out_hbm.at[pl.ds(base + off, CHUNK)], sem_w[b])

    for b in range(NBUF):
        a = bufa.at[pl.ds(b * CHUNK, CHUNK)]
        pltpu.make_async_copy(
            a, out_hbm.at[pl.ds(base + b * CHUNK, CHUNK)], sem_w[b]).wait()


@jax.jit
def kernel(x, edge_index):
    eidx = edge_index.reshape(-1)

    mesh = plsc.VectorSubcoreMesh(core_axis_name="c", subcore_axis_name="s")
    out = pl.kernel(
        _sc_body,
        out_type=jax.ShapeDtypeStruct((N_EDGES, D), jnp.float32),
        mesh=mesh,
        scratch_types=[
            pltpu.VMEM((E_PER_W,), jnp.int32),
            pltpu.VMEM((E_PER_W,), jnp.int32),
            pltpu.VMEM((NBUF * CHUNK, D), jnp.float32),
            pltpu.VMEM_SHARED((N_NODES, D), jnp.float32),
            [pltpu.SemaphoreType.DMA] * NBUF,
            [pltpu.SemaphoreType.DMA] * NBUF,
        ],
    )(x, eidx)
    return out


# speedup vs baseline: 1.0553x; 1.0111x over previous
import jax
import jax.numpy as jnp
from jax import lax
from jax.experimental import pallas as pl
from jax.experimental.pallas import tpu as pltpu
from jax.experimental.pallas import tpu_sc as plsc

N_NODES = 10000
N_EDGES = 320000
D = 128

NC = 2
NS = 16
NW = NC * NS

E_PER_W = N_EDGES // NW
CHUNK = 80
STEPS = E_PER_W // CHUNK
NBUF = 4
GCH = NBUF * CHUNK
GROUPS = E_PER_W // GCH
REM = STEPS - NBUF * GROUPS

STG = 16
ROWS_T = 624
ROWS_LAST = N_NODES - 15 * ROWS_T


def _sc_body(x_hbm, eidx_hbm, out_hbm,
             isrc, idst, bufa, xneg_spm, sem_a, sem_w, sem_i):
    cid = lax.axis_index("c")
    sid = lax.axis_index("s")
    wid = sid * NC + cid
    base = wid * E_PER_W

    ic1 = pltpu.async_copy(eidx_hbm.at[pl.ds(base, GCH)],
                           isrc.at[pl.ds(0, GCH)], sem_i[0])
    ic2 = pltpu.async_copy(eidx_hbm.at[pl.ds(N_EDGES + base, GCH)],
                           idst.at[pl.ds(0, GCH)], sem_i[0])

    rowbase = sid * ROWS_T
    nchunks = jnp.where(sid == NS - 1, ROWS_LAST // STG, ROWS_T // STG)
    tmp = bufa.at[pl.ds(0, STG)]

    def stage(c, carry):
        r = rowbase + c * STG
        pltpu.sync_copy(x_hbm.at[pl.ds(r, STG)], tmp)
        for row in range(STG):
            for jj in range(D // 16):
                s = pl.ds(jj * 16, 16)
                bufa[row, s] = -bufa[row, s]
        pltpu.sync_copy(tmp, xneg_spm.at[pl.ds(r, STG)])
        return carry

    lax.fori_loop(0, nchunks, stage, 0)
    plsc.subcore_barrier()
    ic1.wait()
    ic2.wait()

    def prefetch(g_next, p_next):
        pb = p_next * GCH
        eb = base + g_next * GCH
        pltpu.async_copy(eidx_hbm.at[pl.ds(eb, GCH)],
                         isrc.at[pl.ds(pb, GCH)], sem_i[p_next])
        pltpu.async_copy(eidx_hbm.at[pl.ds(N_EDGES + eb, GCH)],
                         idst.at[pl.ds(pb, GCH)], sem_i[p_next])

    def wait_idx(g, p):
        pb = p * GCH
        eb = base + g * GCH
        pltpu.make_async_copy(eidx_hbm.at[pl.ds(eb, GCH)],
                              isrc.at[pl.ds(pb, GCH)], sem_i[p]).wait()
        pltpu.make_async_copy(eidx_hbm.at[pl.ds(N_EDGES + eb, GCH)],
                              idst.at[pl.ds(pb, GCH)], sem_i[p]).wait()

    def do_group(g, p, first):
        pb = p * GCH
        ga = []
        for b in range(NBUF):
            off = g * GCH + b * CHUNK
            a = bufa.at[pl.ds(b * CHUNK, CHUNK)]
            if not first:
                pltpu.make_async_copy(
                    a, out_hbm.at[pl.ds(base + off, CHUNK)], sem_w[b]).wait()
            ga.append(pltpu.async_copy(
                x_hbm.at[isrc.at[pl.ds(pb + b * CHUNK, CHUNK)]], a,
                sem_a[b]))
        for b in range(NBUF):
            off = g * GCH + b * CHUNK
            a = bufa.at[pl.ds(b * CHUNK, CHUNK)]
            ga[b].wait()
            pltpu.async_copy(
                xneg_spm.at[idst.at[pl.ds(pb + b * CHUNK, CHUNK)]], a,
                sem_a[b], add=True).wait()
            pltpu.async_copy(a, out_hbm.at[pl.ds(base + off, CHUNK)],
                             sem_w[b])

    prefetch(1, 1)
    do_group(0, 0, True)

    def body(gp, carry):
        g = 1 + 2 * gp

        @pl.when(g + 1 < GROUPS)
        def _():
            prefetch(g + 1, 0)

        wait_idx(g, 1)
        do_group(g, 1, False)

        @pl.when(g + 2 < GROUPS)
        def _():
            prefetch(g + 2, 1)

        wait_idx(g + 1, 0)
        do_group(g + 1, 0, False)
        return carry

    lax.fori_loop(0, (GROUPS - 1) // 2, body, 0)

    off = NBUF * GROUPS * CHUNK
    a = bufa.at[pl.ds(0, CHUNK)]
    pltpu.make_async_copy(
        a, out_hbm.at[pl.ds(base + off, CHUNK)], sem_w[0]).wait()
    pltpu.sync_copy(eidx_hbm.at[pl.ds(base + off, CHUNK)],
                    isrc.at[pl.ds(0, CHUNK)])
    pltpu.sync_copy(eidx_hbm.at[pl.ds(N_EDGES + base + off, CHUNK)],
                    idst.at[pl.ds(0, CHUNK)])
    pltpu.async_copy(
        x_hbm.at[isrc.at[pl.ds(0, CHUNK)]], a, sem_a[0]).wait()
    pltpu.async_copy(
        xneg_spm.at[idst.at[pl.ds(0, CHUNK)]], a, sem_a[0], add=True).wait()
    pltpu.async_copy(a, out_hbm.at[pl.ds(base + off, CHUNK)], sem_w[0])

    pltpu.make_async_copy(
        a, out_hbm.at[pl.ds(base + off, CHUNK)], sem_w[0]).wait()
    for b in range(1, NBUF):
        ab = bufa.at[pl.ds(b * CHUNK, CHUNK)]
        pltpu.make_async_copy(
            ab, out_hbm.at[pl.ds(base + b * CHUNK, CHUNK)], sem_w[b]).wait()


@jax.jit
def kernel(x, edge_index):
    eidx = edge_index.reshape(-1)

    mesh = plsc.VectorSubcoreMesh(core_axis_name="c", subcore_axis_name="s")
    out = pl.kernel(
        _sc_body,
        out_type=jax.ShapeDtypeStruct((N_EDGES, D), jnp.float32),
        mesh=mesh,
        scratch_types=[
            pltpu.VMEM((2 * GCH,), jnp.int32),
            pltpu.VMEM((2 * GCH,), jnp.int32),
            pltpu.VMEM((NBUF * CHUNK, D), jnp.float32),
            pltpu.VMEM_SHARED((N_NODES, D), jnp.float32),
            [pltpu.SemaphoreType.DMA] * NBUF,
            [pltpu.SemaphoreType.DMA] * NBUF,
            [pltpu.SemaphoreType.DMA] * 2,
        ],
    )(x, eidx)
    return out


# speedup vs baseline: 1.1455x; 1.0855x over previous
import jax
import jax.numpy as jnp
from jax import lax
from jax.experimental import pallas as pl
from jax.experimental.pallas import tpu as pltpu
from jax.experimental.pallas import tpu_sc as plsc

N_NODES = 10000
N_EDGES = 320000
D = 128

NC = 2
NS = 16
NW = NC * NS

E_PER_W = N_EDGES // NW
CHUNK = 80
STEPS = E_PER_W // CHUNK
NBUF = 4
GCH = NBUF * CHUNK
GROUPS = E_PER_W // GCH
REM = STEPS - NBUF * GROUPS

STG = 48
ROWS_T = 624
TAIL = N_NODES - NS * ROWS_T


def _sc_body(x_hbm, eidx_hbm, out_hbm,
             isrc, idst, bufa, xneg_spm, sem_a, sem_w, sem_i):
    cid = lax.axis_index("c")
    sid = lax.axis_index("s")
    wid = sid * NC + cid
    base = wid * E_PER_W

    ic1 = pltpu.async_copy(eidx_hbm.at[pl.ds(base, GCH)],
                           isrc.at[pl.ds(0, GCH)], sem_i[0])
    ic2 = pltpu.async_copy(eidx_hbm.at[pl.ds(N_EDGES + base, GCH)],
                           idst.at[pl.ds(0, GCH)], sem_i[0])

    rowbase = sid * ROWS_T
    tmp = bufa.at[pl.ds(0, STG)]

    def stage(c, carry):
        r = rowbase + c * STG
        pltpu.sync_copy(x_hbm.at[pl.ds(r, STG)], tmp)
        for row in range(STG):
            for jj in range(D // 16):
                s = pl.ds(jj * 16, 16)
                bufa[row, s] = -bufa[row, s]
        pltpu.sync_copy(tmp, xneg_spm.at[pl.ds(r, STG)])
        return carry

    lax.fori_loop(0, ROWS_T // STG, stage, 0)

    @pl.when(sid == NS - 1)
    def _():
        r = NS * ROWS_T
        tt = bufa.at[pl.ds(0, TAIL)]
        pltpu.sync_copy(x_hbm.at[pl.ds(r, TAIL)], tt)
        for row in range(TAIL):
            for jj in range(D // 16):
                s = pl.ds(jj * 16, 16)
                bufa[row, s] = -bufa[row, s]
        pltpu.sync_copy(tt, xneg_spm.at[pl.ds(r, TAIL)])

    plsc.subcore_barrier()
    ic1.wait()
    ic2.wait()

    def prefetch(g_next, p_next):
        pb = p_next * GCH
        eb = base + g_next * GCH
        pltpu.async_copy(eidx_hbm.at[pl.ds(eb, GCH)],
                         isrc.at[pl.ds(pb, GCH)], sem_i[p_next])
        pltpu.async_copy(eidx_hbm.at[pl.ds(N_EDGES + eb, GCH)],
                         idst.at[pl.ds(pb, GCH)], sem_i[p_next])

    def wait_idx(g, p):
        pb = p * GCH
        eb = base + g * GCH
        pltpu.make_async_copy(eidx_hbm.at[pl.ds(eb, GCH)],
                              isrc.at[pl.ds(pb, GCH)], sem_i[p]).wait()
        pltpu.make_async_copy(eidx_hbm.at[pl.ds(N_EDGES + eb, GCH)],
                              idst.at[pl.ds(pb, GCH)], sem_i[p]).wait()

    def do_group(g, p, first):
        pb = p * GCH
        ga = []
        for b in range(NBUF):
            off = g * GCH + b * CHUNK
            a = bufa.at[pl.ds(b * CHUNK, CHUNK)]
            if not first:
                pltpu.make_async_copy(
                    a, out_hbm.at[pl.ds(base + off, CHUNK)], sem_w[b]).wait()
            ga.append(pltpu.async_copy(
                x_hbm.at[isrc.at[pl.ds(pb + b * CHUNK, CHUNK)]], a,
                sem_a[b]))
        for b in range(NBUF):
            off = g * GCH + b * CHUNK
            a = bufa.at[pl.ds(b * CHUNK, CHUNK)]
            ga[b].wait()
            pltpu.async_copy(
                xneg_spm.at[idst.at[pl.ds(pb + b * CHUNK, CHUNK)]], a,
                sem_a[b], add=True).wait()
            pltpu.async_copy(a, out_hbm.at[pl.ds(base + off, CHUNK)],
                             sem_w[b])

    prefetch(1, 1)
    do_group(0, 0, True)

    def body(gp, carry):
        g = 1 + 2 * gp

        @pl.when(g + 1 < GROUPS)
        def _():
            prefetch(g + 1, 0)

        wait_idx(g, 1)
        do_group(g, 1, False)

        @pl.when(g + 2 < GROUPS)
        def _():
            prefetch(g + 2, 1)

        wait_idx(g + 1, 0)
        do_group(g + 1, 0, False)
        return carry

    lax.fori_loop(0, (GROUPS - 1) // 2, body, 0)

    off = NBUF * GROUPS * CHUNK
    a = bufa.at[pl.ds(0, CHUNK)]
    pltpu.make_async_copy(
        a, out_hbm.at[pl.ds(base + off, CHUNK)], sem_w[0]).wait()
    pltpu.sync_copy(eidx_hbm.at[pl.ds(base + off, CHUNK)],
                    isrc.at[pl.ds(0, CHUNK)])
    pltpu.sync_copy(eidx_hbm.at[pl.ds(N_EDGES + base + off, CHUNK)],
                    idst.at[pl.ds(0, CHUNK)])
    pltpu.async_copy(
        x_hbm.at[isrc.at[pl.ds(0, CHUNK)]], a, sem_a[0]).wait()
    pltpu.async_copy(
        xneg_spm.at[idst.at[pl.ds(0, CHUNK)]], a, sem_a[0], add=True).wait()
    pltpu.async_copy(a, out_hbm.at[pl.ds(base + off, CHUNK)], sem_w[0])

    pltpu.make_async_copy(
        a, out_hbm.at[pl.ds(base + off, CHUNK)], sem_w[0]).wait()
    for b in range(1, NBUF):
        ab = bufa.at[pl.ds(b * CHUNK, CHUNK)]
        pltpu.make_async_copy(
            ab, out_hbm.at[pl.ds(base + b * CHUNK, CHUNK)], sem_w[b]).wait()


@jax.jit
def kernel(x, edge_index):
    eidx = edge_index.reshape(-1)

    mesh = plsc.VectorSubcoreMesh(core_axis_name="c", subcore_axis_name="s")
    out = pl.kernel(
        _sc_body,
        out_type=jax.ShapeDtypeStruct((N_EDGES, D), jnp.float32),
        mesh=mesh,
        scratch_types=[
            pltpu.VMEM((2 * GCH,), jnp.int32),
            pltpu.VMEM((2 * GCH,), jnp.int32),
            pltpu.VMEM((NBUF * CHUNK, D), jnp.float32),
            pltpu.VMEM_SHARED((N_NODES, D), jnp.float32),
            [pltpu.SemaphoreType.DMA] * NBUF,
            [pltpu.SemaphoreType.DMA] * NBUF,
            [pltpu.SemaphoreType.DMA] * 2,
        ],
    )(x, eidx)
    return out


# speedup vs baseline: 1.1751x; 1.0259x over previous
import jax
import jax.numpy as jnp
from jax import lax
from jax.experimental import pallas as pl
from jax.experimental.pallas import tpu as pltpu
from jax.experimental.pallas import tpu_sc as plsc

N_NODES = 10000
N_EDGES = 320000
D = 128

NC = 2
NS = 16
NW = NC * NS

E_PER_W = N_EDGES // NW
CHUNK = 80
STEPS = E_PER_W // CHUNK
NBUF = 4
GCH = NBUF * CHUNK
GROUPS = E_PER_W // GCH
REM = STEPS - NBUF * GROUPS

STG = 104
ROWS_T = 624
TAIL = N_NODES - NS * ROWS_T


def _sc_body(x_hbm, eidx_hbm, out_hbm,
             isrc, idst, bufa, xneg_spm, sem_a, sem_w, sem_i):
    cid = lax.axis_index("c")
    sid = lax.axis_index("s")
    wid = sid * NC + cid
    base = wid * E_PER_W

    ic1 = pltpu.async_copy(eidx_hbm.at[pl.ds(base, GCH)],
                           isrc.at[pl.ds(0, GCH)], sem_i[0])
    ic2 = pltpu.async_copy(eidx_hbm.at[pl.ds(N_EDGES + base, GCH)],
                           idst.at[pl.ds(0, GCH)], sem_i[0])

    rowbase = sid * ROWS_T
    regs = [bufa.at[pl.ds(0, STG)], bufa.at[pl.ds(STG, STG)]]

    def st_pf(c, p):
        return pltpu.async_copy(
            x_hbm.at[pl.ds(rowbase + c * STG, STG)], regs[p], sem_a[p])

    def st_pf_wait(c, p):
        pltpu.make_async_copy(
            x_hbm.at[pl.ds(rowbase + c * STG, STG)], regs[p],
            sem_a[p]).wait()

    def st_wb(c, p):
        return pltpu.async_copy(
            regs[p], xneg_spm.at[pl.ds(rowbase + c * STG, STG)], sem_w[p])

    def st_wb_wait(c, p):
        pltpu.make_async_copy(
            regs[p], xneg_spm.at[pl.ds(rowbase + c * STG, STG)],
            sem_w[p]).wait()

    def st_neg(p):
        r0 = p * STG
        for row in range(STG):
            for jj in range(D // 16):
                s = pl.ds(jj * 16, 16)
                bufa[r0 + row, s] = -bufa[r0 + row, s]

    NPAIR = ROWS_T // STG // 2
    st_pf(0, 0)
    st_pf(1, 1)

    def st_pair(g, carry):
        c0 = 2 * g
        st_pf_wait(c0, 0)
        st_neg(0)
        st_wb(c0, 0)
        st_pf_wait(c0 + 1, 1)
        st_neg(1)
        st_wb(c0 + 1, 1)

        @pl.when(g < NPAIR - 1)
        def _():
            st_wb_wait(c0, 0)
            st_pf(c0 + 2, 0)
            st_wb_wait(c0 + 1, 1)
            st_pf(c0 + 3, 1)

        return carry

    lax.fori_loop(0, NPAIR, st_pair, 0)
    st_wb_wait(ROWS_T // STG - 2, 0)
    st_wb_wait(ROWS_T // STG - 1, 1)

    @pl.when(sid == NS - 1)
    def _():
        r = NS * ROWS_T
        tt = bufa.at[pl.ds(0, TAIL)]
        pltpu.sync_copy(x_hbm.at[pl.ds(r, TAIL)], tt)
        for row in range(TAIL):
            for jj in range(D // 16):
                s = pl.ds(jj * 16, 16)
                bufa[row, s] = -bufa[row, s]
        pltpu.sync_copy(tt, xneg_spm.at[pl.ds(r, TAIL)])

    plsc.subcore_barrier()
    ic1.wait()
    ic2.wait()

    def prefetch(g_next, p_next):
        pb = p_next * GCH
        eb = base + g_next * GCH
        pltpu.async_copy(eidx_hbm.at[pl.ds(eb, GCH)],
                         isrc.at[pl.ds(pb, GCH)], sem_i[p_next])
        pltpu.async_copy(eidx_hbm.at[pl.ds(N_EDGES + eb, GCH)],
                         idst.at[pl.ds(pb, GCH)], sem_i[p_next])

    def wait_idx(g, p):
        pb = p * GCH
        eb = base + g * GCH
        pltpu.make_async_copy(eidx_hbm.at[pl.ds(eb, GCH)],
                              isrc.at[pl.ds(pb, GCH)], sem_i[p]).wait()
        pltpu.make_async_copy(eidx_hbm.at[pl.ds(N_EDGES + eb, GCH)],
                              idst.at[pl.ds(pb, GCH)], sem_i[p]).wait()

    def do_group(g, p, first):
        pb = p * GCH
        ga = []
        for b in range(NBUF):
            off = g * GCH + b * CHUNK
            a = bufa.at[pl.ds(b * CHUNK, CHUNK)]
            if not first:
                pltpu.make_async_copy(
                    a, out_hbm.at[pl.ds(base + off, CHUNK)], sem_w[b]).wait()
            ga.append(pltpu.async_copy(
                x_hbm.at[isrc.at[pl.ds(pb + b * CHUNK, CHUNK)]], a,
                sem_a[b]))
        for b in range(NBUF):
            off = g * GCH + b * CHUNK
            a = bufa.at[pl.ds(b * CHUNK, CHUNK)]
            ga[b].wait()
            pltpu.async_copy(
                xneg_spm.at[idst.at[pl.ds(pb + b * CHUNK, CHUNK)]], a,
                sem_a[b], add=True).wait()
            pltpu.async_copy(a, out_hbm.at[pl.ds(base + off, CHUNK)],
                             sem_w[b])

    prefetch(1, 1)
    do_group(0, 0, True)

    def body(gp, carry):
        g = 1 + 2 * gp

        @pl.when(g + 1 < GROUPS)
        def _():
            prefetch(g + 1, 0)

        wait_idx(g, 1)
        do_group(g, 1, False)

        @pl.when(g + 2 < GROUPS)
        def _():
            prefetch(g + 2, 1)

        wait_idx(g + 1, 0)
        do_group(g + 1, 0, False)
        return carry

    lax.fori_loop(0, (GROUPS - 1) // 2, body, 0)

    off = NBUF * GROUPS * CHUNK
    a = bufa.at[pl.ds(0, CHUNK)]
    pltpu.make_async_copy(
        a, out_hbm.at[pl.ds(base + off, CHUNK)], sem_w[0]).wait()
    pltpu.sync_copy(eidx_hbm.at[pl.ds(base + off, CHUNK)],
                    isrc.at[pl.ds(0, CHUNK)])
    pltpu.sync_copy(eidx_hbm.at[pl.ds(N_EDGES + base + off, CHUNK)],
                    idst.at[pl.ds(0, CHUNK)])
    pltpu.async_copy(
        x_hbm.at[isrc.at[pl.ds(0, CHUNK)]], a, sem_a[0]).wait()
    pltpu.async_copy(
        xneg_spm.at[idst.at[pl.ds(0, CHUNK)]], a, sem_a[0], add=True).wait()
    pltpu.async_copy(a, out_hbm.at[pl.ds(base + off, CHUNK)], sem_w[0])

    pltpu.make_async_copy(
        a, out_hbm.at[pl.ds(base + off, CHUNK)], sem_w[0]).wait()
    for b in range(1, NBUF):
        ab = bufa.at[pl.ds(b * CHUNK, CHUNK)]
        pltpu.make_async_copy(
            ab, out_hbm.at[pl.ds(base + b * CHUNK, CHUNK)], sem_w[b]).wait()


@jax.jit
def kernel(x, edge_index):
    eidx = edge_index.reshape(-1)

    mesh = plsc.VectorSubcoreMesh(core_axis_name="c", subcore_axis_name="s")
    out = pl.kernel(
        _sc_body,
        out_type=jax.ShapeDtypeStruct((N_EDGES, D), jnp.float32),
        mesh=mesh,
        scratch_types=[
            pltpu.VMEM((2 * GCH,), jnp.int32),
            pltpu.VMEM((2 * GCH,), jnp.int32),
            pltpu.VMEM((NBUF * CHUNK, D), jnp.float32),
            pltpu.VMEM_SHARED((N_NODES, D), jnp.float32),
            [pltpu.SemaphoreType.DMA] * NBUF,
            [pltpu.SemaphoreType.DMA] * NBUF,
            [pltpu.SemaphoreType.DMA] * 2,
        ],
    )(x, eidx)
    return out


# speedup vs baseline: 1.2009x; 1.0220x over previous
import jax
import jax.numpy as jnp
from jax import lax
from jax.experimental import pallas as pl
from jax.experimental.pallas import tpu as pltpu
from jax.experimental.pallas import tpu_sc as plsc

N_NODES = 10000
N_EDGES = 320000
D = 128

NC = 2
NS = 16
NW = NC * NS

E_PER_W = N_EDGES // NW
CHUNK = 96
NBUF = 4
GCH = NBUF * CHUNK
GROUPS = E_PER_W // GCH
RCH = E_PER_W - GROUPS * GCH

STG = 104
ROWS_T = 624
TAIL = N_NODES - NS * ROWS_T


def _sc_body(x_hbm, eidx_hbm, out_hbm,
             isrc, idst, bufa, xneg_spm, sem_a, sem_w, sem_i):
    cid = lax.axis_index("c")
    sid = lax.axis_index("s")
    wid = sid * NC + cid
    base = wid * E_PER_W

    ic1 = pltpu.async_copy(eidx_hbm.at[pl.ds(base, GCH)],
                           isrc.at[pl.ds(0, GCH)], sem_i[0])
    ic2 = pltpu.async_copy(eidx_hbm.at[pl.ds(N_EDGES + base, GCH)],
                           idst.at[pl.ds(0, GCH)], sem_i[0])

    rowbase = sid * ROWS_T
    regs = [bufa.at[pl.ds(0, STG)], bufa.at[pl.ds(STG, STG)]]

    def st_pf(c, p):
        return pltpu.async_copy(
            x_hbm.at[pl.ds(rowbase + c * STG, STG)], regs[p], sem_a[p])

    def st_pf_wait(c, p):
        pltpu.make_async_copy(
            x_hbm.at[pl.ds(rowbase + c * STG, STG)], regs[p],
            sem_a[p]).wait()

    def st_wb(c, p):
        return pltpu.async_copy(
            regs[p], xneg_spm.at[pl.ds(rowbase + c * STG, STG)], sem_w[p])

    def st_wb_wait(c, p):
        pltpu.make_async_copy(
            regs[p], xneg_spm.at[pl.ds(rowbase + c * STG, STG)],
            sem_w[p]).wait()

    def st_neg(p):
        r0 = p * STG
        for row in range(STG):
            for jj in range(D // 16):
                s = pl.ds(jj * 16, 16)
                bufa[r0 + row, s] = -bufa[r0 + row, s]

    NPAIR = ROWS_T // STG // 2
    st_pf(0, 0)
    st_pf(1, 1)

    def st_pair(g, carry):
        c0 = 2 * g
        st_pf_wait(c0, 0)
        st_neg(0)
        st_wb(c0, 0)
        st_pf_wait(c0 + 1, 1)
        st_neg(1)
        st_wb(c0 + 1, 1)

        @pl.when(g < NPAIR - 1)
        def _():
            st_wb_wait(c0, 0)
            st_pf(c0 + 2, 0)
            st_wb_wait(c0 + 1, 1)
            st_pf(c0 + 3, 1)

        return carry

    lax.fori_loop(0, NPAIR, st_pair, 0)
    st_wb_wait(ROWS_T // STG - 2, 0)
    st_wb_wait(ROWS_T // STG - 1, 1)

    @pl.when(sid == NS - 1)
    def _():
        r = NS * ROWS_T
        tt = bufa.at[pl.ds(0, TAIL)]
        pltpu.sync_copy(x_hbm.at[pl.ds(r, TAIL)], tt)
        for row in range(TAIL):
            for jj in range(D // 16):
                s = pl.ds(jj * 16, 16)
                bufa[row, s] = -bufa[row, s]
        pltpu.sync_copy(tt, xneg_spm.at[pl.ds(r, TAIL)])

    plsc.subcore_barrier()
    ic1.wait()
    ic2.wait()

    def prefetch(g_next, p_next):
        pb = p_next * GCH
        eb = base + g_next * GCH
        pltpu.async_copy(eidx_hbm.at[pl.ds(eb, GCH)],
                         isrc.at[pl.ds(pb, GCH)], sem_i[p_next])
        pltpu.async_copy(eidx_hbm.at[pl.ds(N_EDGES + eb, GCH)],
                         idst.at[pl.ds(pb, GCH)], sem_i[p_next])

    def wait_idx(g, p):
        pb = p * GCH
        eb = base + g * GCH
        pltpu.make_async_copy(eidx_hbm.at[pl.ds(eb, GCH)],
                              isrc.at[pl.ds(pb, GCH)], sem_i[p]).wait()
        pltpu.make_async_copy(eidx_hbm.at[pl.ds(N_EDGES + eb, GCH)],
                              idst.at[pl.ds(pb, GCH)], sem_i[p]).wait()

    def do_group(g, p, first):
        pb = p * GCH
        ga = []
        for b in range(NBUF):
            off = g * GCH + b * CHUNK
            a = bufa.at[pl.ds(b * CHUNK, CHUNK)]
            if not first:
                pltpu.make_async_copy(
                    a, out_hbm.at[pl.ds(base + off, CHUNK)], sem_w[b]).wait()
            ga.append(pltpu.async_copy(
                x_hbm.at[isrc.at[pl.ds(pb + b * CHUNK, CHUNK)]], a,
                sem_a[b]))
        for b in range(NBUF):
            off = g * GCH + b * CHUNK
            a = bufa.at[pl.ds(b * CHUNK, CHUNK)]
            ga[b].wait()
            pltpu.async_copy(
                xneg_spm.at[idst.at[pl.ds(pb + b * CHUNK, CHUNK)]], a,
                sem_a[b], add=True).wait()
            pltpu.async_copy(a, out_hbm.at[pl.ds(base + off, CHUNK)],
                             sem_w[b])

    prefetch(1, 1)
    do_group(0, 0, True)

    def body(gp, carry):
        g = 1 + 2 * gp

        @pl.when(g + 1 < GROUPS)
        def _():
            prefetch(g + 1, 0)

        wait_idx(g, 1)
        do_group(g, 1, False)

        @pl.when(g + 2 < GROUPS)
        def _():
            prefetch(g + 2, 1)

        wait_idx(g + 1, 0)
        do_group(g + 1, 0, False)
        return carry

    lax.fori_loop(0, (GROUPS - 1) // 2, body, 0)

    if (GROUPS - 1) % 2 == 1:
        gl = GROUPS - 1
        wait_idx(gl, gl % 2)
        do_group(gl, gl % 2, False)

    off = GROUPS * GCH
    a = bufa.at[pl.ds(0, RCH)]
    pltpu.make_async_copy(
        bufa.at[pl.ds(0, CHUNK)], out_hbm.at[pl.ds(base, CHUNK)],
        sem_w[0]).wait()
    pltpu.sync_copy(eidx_hbm.at[pl.ds(base + off, RCH)],
                    isrc.at[pl.ds(0, RCH)])
    pltpu.sync_copy(eidx_hbm.at[pl.ds(N_EDGES + base + off, RCH)],
                    idst.at[pl.ds(0, RCH)])
    pltpu.async_copy(
        x_hbm.at[isrc.at[pl.ds(0, RCH)]], a, sem_a[0]).wait()
    pltpu.async_copy(
        xneg_spm.at[idst.at[pl.ds(0, RCH)]], a, sem_a[0], add=True).wait()
    pltpu.async_copy(a, out_hbm.at[pl.ds(base + off, RCH)], sem_w[0])

    pltpu.make_async_copy(
        a, out_hbm.at[pl.ds(base + off, RCH)], sem_w[0]).wait()
    for b in range(1, NBUF):
        ab = bufa.at[pl.ds(b * CHUNK, CHUNK)]
        pltpu.make_async_copy(
            ab, out_hbm.at[pl.ds(base + b * CHUNK, CHUNK)], sem_w[b]).wait()


@jax.jit
def kernel(x, edge_index):
    eidx = edge_index.reshape(-1)

    mesh = plsc.VectorSubcoreMesh(core_axis_name="c", subcore_axis_name="s")
    out = pl.kernel(
        _sc_body,
        out_type=jax.ShapeDtypeStruct((N_EDGES, D), jnp.float32),
        mesh=mesh,
        scratch_types=[
            pltpu.VMEM((2 * GCH,), jnp.int32),
            pltpu.VMEM((2 * GCH,), jnp.int32),
            pltpu.VMEM((NBUF * CHUNK, D), jnp.float32),
            pltpu.VMEM_SHARED((N_NODES, D), jnp.float32),
            [pltpu.SemaphoreType.DMA] * NBUF,
            [pltpu.SemaphoreType.DMA] * NBUF,
            [pltpu.SemaphoreType.DMA] * 2,
        ],
    )(x, eidx)
    return out
